# baseline (speedup 1.0000x reference)
import functools

import jax
import jax.numpy as jnp
from jax import lax
from jax.experimental import pallas as pl
from jax.experimental.pallas import tpu as pltpu
from jax.experimental.pallas import tpu_sc as plsc

_L = 16


def _prep_body(h_hbm, wt_ref, b_ref, ei_ref, pq_ref, src_ref, dst_ref,
               h_vmem, h_sem):
    cp = pltpu.make_async_copy(h_hbm, h_vmem, h_sem)
    cp.start()
    src_ref[...] = ei_ref[0, :]
    dst_ref[...] = ei_ref[1, :]
    cp.wait()
    pq = jax.lax.dot_general(
        wt_ref[...], h_vmem[...], (((1,), (1,)), ((), ())),
        preferred_element_type=jnp.float32)
    pq_ref[0:1, :] = pq[0:1, :] + b_ref[0]
    pq_ref[1:2, :] = pq[1:2, :]


@functools.cache
def _make_sc_gather(n_nodes, n_edges, nc, ns):
    nw = nc * ns
    epw = n_edges // nw
    mesh = plsc.VectorSubcoreMesh(core_axis_name="c", subcore_axis_name="s")

    @functools.partial(
        pl.kernel,
        mesh=mesh,
        compiler_params=pltpu.CompilerParams(
            needs_layout_passes=False, skip_device_barrier=True),
        out_type=jax.ShapeDtypeStruct((n_edges,), jnp.float32),
        scratch_types=[
            pltpu.VMEM((n_nodes,), jnp.float32),
            pltpu.VMEM((n_nodes,), jnp.float32),
            pltpu.VMEM((epw,), jnp.int32),
            pltpu.VMEM((epw,), jnp.int32),
            pltpu.VMEM((epw,), jnp.float32),
            pltpu.SemaphoreType.DMA,
        ],
    )
    def sc_gather(pq_hbm, src_hbm, dst_hbm, out_hbm,
                  p_v, q_v, src_v, dst_v, out_v, sem):
        wid = lax.axis_index("s") * nc + lax.axis_index("c")
        base = wid * epw
        c1 = pltpu.make_async_copy(pq_hbm.at[0], p_v, sem)
        c2 = pltpu.make_async_copy(pq_hbm.at[1], q_v, sem)
        c3 = pltpu.make_async_copy(src_hbm.at[pl.ds(base, epw)], src_v, sem)
        c4 = pltpu.make_async_copy(dst_hbm.at[pl.ds(base, epw)], dst_v, sem)
        c1.start(); c2.start(); c3.start(); c4.start()
        c1.wait(); c2.wait(); c3.wait(); c4.wait()

        unroll = 5

        def body(i, carry):
            for j in range(unroll):
                off = (i * unroll + j) * _L
                s16 = src_v[pl.ds(off, _L)]
                d16 = dst_v[pl.ds(off, _L)]
                vals = (plsc.load_gather(p_v, [s16])
                        + plsc.load_gather(q_v, [d16]))
                out_v[pl.ds(off, _L)] = vals
            return carry

        lax.fori_loop(0, epw // (_L * unroll), body, 0)
        pltpu.sync_copy(out_v, out_hbm.at[pl.ds(base, epw)])

    return sc_gather


def kernel(h, edge_index, W, b):
    n_nodes, d = h.shape
    n_edges = edge_index.shape[1]
    ei = edge_index.astype(jnp.int32)
    wt = W.reshape(2, d)

    pq, src, dst = pl.pallas_call(
        _prep_body,
        in_specs=[
            pl.BlockSpec(memory_space=pl.ANY),
            pl.BlockSpec(memory_space=pltpu.VMEM),
            pl.BlockSpec(memory_space=pltpu.SMEM),
            pl.BlockSpec(memory_space=pltpu.VMEM),
        ],
        out_specs=[
            pl.BlockSpec(memory_space=pltpu.VMEM),
            pl.BlockSpec(memory_space=pltpu.VMEM),
            pl.BlockSpec(memory_space=pltpu.VMEM),
        ],
        scratch_shapes=[
            pltpu.VMEM((n_nodes, d), jnp.float32),
            pltpu.SemaphoreType.DMA,
        ],
        out_shape=[
            jax.ShapeDtypeStruct((2, n_nodes), jnp.float32),
            jax.ShapeDtypeStruct((n_edges,), jnp.int32),
            jax.ShapeDtypeStruct((n_edges,), jnp.int32),
        ],
    )(h, wt, b.astype(jnp.float32), ei)

    info = plsc.get_sparse_core_info()
    sc = _make_sc_gather(n_nodes, n_edges, info.num_cores, info.num_subcores)
    return sc(pq, src, dst).reshape(n_edges, 1)


# speedup vs baseline: 1.0669x; 1.0669x over previous
import functools

import jax
import jax.numpy as jnp
from jax import lax
from jax.experimental import pallas as pl
from jax.experimental.pallas import tpu as pltpu
from jax.experimental.pallas import tpu_sc as plsc

_L = 16


def _prep_body(h_hbm, wt_ref, b_ref, ei_ref, pq_ref, src_ref, dst_ref,
               h_vmem, h_sem):
    cp = pltpu.make_async_copy(h_hbm, h_vmem, h_sem)
    cp.start()
    src_ref[...] = ei_ref[0, :]
    dst_ref[...] = ei_ref[1, :]
    cp.wait()
    pq = jax.lax.dot_general(
        wt_ref[...], h_vmem[...], (((1,), (1,)), ((), ())),
        preferred_element_type=jnp.float32)
    pq_ref[0:1, :] = pq[0:1, :] + b_ref[0]
    pq_ref[1:2, :] = pq[1:2, :]


@functools.cache
def _make_sc_gather(n_nodes, n_edges, nc, ns):
    nw = nc * ns
    epw = n_edges // nw
    mesh = plsc.VectorSubcoreMesh(core_axis_name="c", subcore_axis_name="s")

    @functools.partial(
        pl.kernel,
        mesh=mesh,
        compiler_params=pltpu.CompilerParams(
            needs_layout_passes=False, skip_device_barrier=True),
        out_type=jax.ShapeDtypeStruct((n_edges,), jnp.float32),
        scratch_types=[
            pltpu.VMEM((n_nodes,), jnp.float32),
            pltpu.VMEM((n_nodes,), jnp.float32),
            pltpu.VMEM((epw,), jnp.int32),
            pltpu.VMEM((epw,), jnp.int32),
            pltpu.VMEM((epw,), jnp.float32),
            pltpu.SemaphoreType.DMA,
        ],
    )
    def sc_gather(pq_hbm, src_hbm, dst_hbm, out_hbm,
                  p_v, q_v, src_v, dst_v, out_v, sem):
        wid = lax.axis_index("s") * nc + lax.axis_index("c")
        base = wid * epw
        c1 = pltpu.make_async_copy(pq_hbm.at[0], p_v, sem)
        c2 = pltpu.make_async_copy(pq_hbm.at[1], q_v, sem)
        c3 = pltpu.make_async_copy(src_hbm.at[pl.ds(base, epw)], src_v, sem)
        c4 = pltpu.make_async_copy(dst_hbm.at[pl.ds(base, epw)], dst_v, sem)
        c1.start(); c2.start(); c3.start(); c4.start()
        c1.wait(); c2.wait(); c3.wait(); c4.wait()

        @plsc.parallel_loop(0, epw // _L, 1, unroll=5)
        def _gather(i):
            off = i * _L
            s16 = src_v[pl.ds(off, _L)]
            d16 = dst_v[pl.ds(off, _L)]
            vals = (plsc.load_gather(p_v, [s16])
                    + plsc.load_gather(q_v, [d16]))
            out_v[pl.ds(off, _L)] = vals
        pltpu.sync_copy(out_v, out_hbm.at[pl.ds(base, epw)])

    return sc_gather


def kernel(h, edge_index, W, b):
    n_nodes, d = h.shape
    n_edges = edge_index.shape[1]
    ei = edge_index.astype(jnp.int32)
    wt = W.reshape(2, d)

    pq, src, dst = pl.pallas_call(
        _prep_body,
        in_specs=[
            pl.BlockSpec(memory_space=pl.ANY),
            pl.BlockSpec(memory_space=pltpu.VMEM),
            pl.BlockSpec(memory_space=pltpu.SMEM),
            pl.BlockSpec(memory_space=pltpu.VMEM),
        ],
        out_specs=[
            pl.BlockSpec(memory_space=pltpu.VMEM),
            pl.BlockSpec(memory_space=pltpu.VMEM),
            pl.BlockSpec(memory_space=pltpu.VMEM),
        ],
        scratch_shapes=[
            pltpu.VMEM((n_nodes, d), jnp.float32),
            pltpu.SemaphoreType.DMA,
        ],
        out_shape=[
            jax.ShapeDtypeStruct((2, n_nodes), jnp.float32),
            jax.ShapeDtypeStruct((n_edges,), jnp.int32),
            jax.ShapeDtypeStruct((n_edges,), jnp.int32),
        ],
    )(h, wt, b.astype(jnp.float32), ei)

    info = plsc.get_sparse_core_info()
    sc = _make_sc_gather(n_nodes, n_edges, info.num_cores, info.num_subcores)
    return sc(pq, src, dst).reshape(n_edges, 1)
